# baseline (speedup 1.0000x reference)
import functools

import jax
import jax.numpy as jnp
from jax import lax
from jax.experimental import pallas as pl
from jax.experimental.pallas import tpu as pltpu
from jax.experimental.pallas import tpu_sc as plsc

N_NODES = 10000
D_IN = 128
D_HID = 128
N_EDGES = 320000
ALPHA = 0.1
K_PROP = 10

NT = 16
NPT = 640
N_PAD = NT * NPT
CHUNK = 128
CPT = 160
E_PAD = NT * CPT * CHUNK
N_VR = NPT // 16

MLP_BLK = 1000


def _dot(a, b):
    return jax.lax.dot(a, b, precision=jax.lax.Precision.HIGHEST)


def _mlp_body(x_ref, w1_ref, b1_ref, w2_ref, b2_ref, wo_ref, z_ref):
    h = jnp.maximum(_dot(x_ref[...], w1_ref[...]) + b1_ref[...], 0.0)
    h = jnp.maximum(_dot(h, w2_ref[...]) + b2_ref[...], 0.0)
    z_ref[...] = _dot(h, wo_ref[...])


def _mlp_project(x, W1, b1, W2, b2, Wo):
    grid = N_NODES // MLP_BLK
    return pl.pallas_call(
        _mlp_body,
        grid=(grid,),
        in_specs=[
            pl.BlockSpec((MLP_BLK, D_IN), lambda i: (i, 0)),
            pl.BlockSpec((D_IN, D_HID), lambda i: (0, 0)),
            pl.BlockSpec((1, D_HID), lambda i: (0, 0)),
            pl.BlockSpec((D_HID, D_HID), lambda i: (0, 0)),
            pl.BlockSpec((1, D_HID), lambda i: (0, 0)),
            pl.BlockSpec((D_HID, 1), lambda i: (0, 0)),
        ],
        out_specs=pl.BlockSpec((MLP_BLK, 1), lambda i: (i, 0)),
        out_shape=jax.ShapeDtypeStruct((N_NODES, 1), jnp.float32),
    )(x, W1, b1.reshape(1, D_HID), W2, b2.reshape(1, D_HID), Wo)


def _rsqrt16(d):
    bi = lax.bitcast_convert_type(d, jnp.int32)
    y = lax.bitcast_convert_type(jnp.int32(0x5F3759DF) - (bi >> 1),
                                 jnp.float32)
    for _ in range(3):
        y = y * (1.5 - 0.5 * d * y * y)
    return y


FK = 16


def _deg_body(dsts_hbm, dinv_hbm, agg_sh, dst_v, tmp_v, ones_v, ssem):
    tid = lax.axis_index("s")
    base = tid * NPT

    pltpu.sync_copy(dsts_hbm.at[pl.ds(tid * CPT, CPT)], dst_v)
    one = jnp.full((16,), 1.0, jnp.float32)
    for i in range(8):
        ones_v[pl.ds(i * 16, 16)] = one
    for i in range(N_VR):
        tmp_v[pl.ds(i * 16, 16)] = one
    pltpu.sync_copy(tmp_v, agg_sh.at[pl.ds(base, NPT)])
    plsc.subcore_barrier()

    @pl.loop(0, CPT, step=FK)
    def _deg(c0):
        for b in range(FK):
            pltpu.async_copy(ones_v, agg_sh.at[dst_v.at[c0 + b]], ssem,
                             add=True)

    pltpu.make_async_copy(dsts_hbm.at[pl.ds(tid * CPT, CPT)], dst_v,
                          ssem).wait()

    plsc.subcore_barrier()
    pltpu.sync_copy(agg_sh.at[pl.ds(base, NPT)], tmp_v)
    for i in range(N_VR):
        sl = pl.ds(i * 16, 16)
        tmp_v[sl] = _rsqrt16(tmp_v[sl])
    pltpu.sync_copy(tmp_v, dinv_hbm.at[pl.ds(base, NPT)])


def _prop_body(z0_hbm, srcs_hbm, dsts_hbm, dinv_hbm, bo_hbm, out_hbm,
               u_sh, agg_sh, src_v, dst_v, z0_v, z_v, dinv_v, tmp_v,
               vals_v, ones_v, bo_v, gsem, ssem):
    tid = lax.axis_index("s")
    base = tid * NPT

    pltpu.sync_copy(srcs_hbm.at[pl.ds(tid * CPT, CPT)], src_v)
    pltpu.sync_copy(dsts_hbm.at[pl.ds(tid * CPT, CPT)], dst_v)
    pltpu.sync_copy(z0_hbm.at[pl.ds(base, NPT)], z0_v)
    pltpu.sync_copy(dinv_hbm.at[pl.ds(base, NPT)], dinv_v)
    pltpu.sync_copy(bo_hbm, bo_v)

    for i in range(N_VR):
        sl = pl.ds(i * 16, 16)
        z_v[sl] = z0_v[sl]

    for _ in range(K_PROP):
        for i in range(N_VR):
            sl = pl.ds(i * 16, 16)
            tmp_v[sl] = dinv_v[sl] * z_v[sl]
        pltpu.sync_copy(tmp_v, u_sh.at[pl.ds(base, NPT)])
        pltpu.sync_copy(tmp_v, agg_sh.at[pl.ds(base, NPT)])
        plsc.subcore_barrier()

        @pl.loop(0, CPT, step=FK)
        def _edges(c0):
            gs = [pltpu.async_copy(u_sh.at[src_v.at[c0 + b]],
                                   vals_v.at[c0 + b], gsem)
                  for b in range(FK)]
            for d in gs:
                d.wait()
            for b in range(FK):
                pltpu.async_copy(vals_v.at[c0 + b],
                                 agg_sh.at[dst_v.at[c0 + b]], ssem,
                                 add=True)

        pltpu.make_async_copy(srcs_hbm.at[pl.ds(tid * CPT, CPT)], src_v,
                              ssem).wait()

        plsc.subcore_barrier()
        pltpu.sync_copy(agg_sh.at[pl.ds(base, NPT)], tmp_v)
        for i in range(N_VR):
            sl = pl.ds(i * 16, 16)
            z_v[sl] = ((1.0 - ALPHA) * dinv_v[sl] * tmp_v[sl]
                       + ALPHA * z0_v[sl])

    bo_vec = bo_v[...]
    for i in range(N_VR):
        sl = pl.ds(i * 16, 16)
        tmp_v[sl] = z_v[sl] + bo_vec
    pltpu.sync_copy(tmp_v, out_hbm.at[pl.ds(base, NPT)])


_deg = pl.kernel(
    _deg_body,
    out_type=jax.ShapeDtypeStruct((N_PAD,), jnp.float32),
    mesh=plsc.VectorSubcoreMesh(
        core_axis_name="c", subcore_axis_name="s", num_cores=1),
    scratch_types=[
        pltpu.VMEM_SHARED((N_PAD,), jnp.float32),
        pltpu.VMEM((CPT, CHUNK), jnp.int32),
        pltpu.VMEM((NPT,), jnp.float32),
        pltpu.VMEM((128,), jnp.float32),
        pltpu.SemaphoreType.DMA,
    ],
)

_prop = pl.kernel(
    _prop_body,
    out_type=jax.ShapeDtypeStruct((N_PAD,), jnp.float32),
    mesh=plsc.VectorSubcoreMesh(
        core_axis_name="c", subcore_axis_name="s", num_cores=1),
    scratch_types=[
        pltpu.VMEM_SHARED((N_PAD,), jnp.float32),
        pltpu.VMEM_SHARED((N_PAD,), jnp.float32),
        pltpu.VMEM((CPT, CHUNK), jnp.int32),
        pltpu.VMEM((CPT, CHUNK), jnp.int32),
        pltpu.VMEM((NPT,), jnp.float32),
        pltpu.VMEM((NPT,), jnp.float32),
        pltpu.VMEM((NPT,), jnp.float32),
        pltpu.VMEM((NPT,), jnp.float32),
        pltpu.VMEM((CPT, CHUNK), jnp.float32),
        pltpu.VMEM((128,), jnp.float32),
        pltpu.VMEM((16,), jnp.float32),
        pltpu.SemaphoreType.DMA,
        pltpu.SemaphoreType.DMA,
    ],
)


@jax.jit
def kernel(x, edge_index, W1, b1, W2, b2, Wo, bo):
    z0 = _mlp_project(x, W1, b1, W2, b2, Wo)[:, 0]
    z0p = jnp.pad(z0, (0, N_PAD - N_NODES))

    ei = edge_index.astype(jnp.int32)
    pad_idx = N_NODES + (jnp.arange(E_PAD - N_EDGES, dtype=jnp.int32)
                         % (N_PAD - N_NODES))
    srcs = jnp.concatenate([ei[0], pad_idx]).reshape(NT * CPT, CHUNK)
    dsts = jnp.concatenate([ei[1], pad_idx]).reshape(NT * CPT, CHUNK)
    bo16 = jnp.broadcast_to(bo.astype(jnp.float32), (16,))

    dinv = _deg(dsts)
    out = _prop(z0p, srcs, dsts, dinv, bo16)
    return out[:N_NODES]


# speedup vs baseline: 1.1509x; 1.1509x over previous
import functools

import jax
import jax.numpy as jnp
from jax import lax
from jax.experimental import pallas as pl
from jax.experimental.pallas import tpu as pltpu
from jax.experimental.pallas import tpu_sc as plsc

N_NODES = 10000
D_IN = 128
D_HID = 128
N_EDGES = 320000
ALPHA = 0.1
K_PROP = 10

NT = 16
NPT = 640
N_PAD = NT * NPT
CHUNK = 128
CPT = 160
E_PAD = NT * CPT * CHUNK
N_VR = NPT // 16

MLP_BLK = 1000


def _dot(a, b):
    return jax.lax.dot(a, b)


def _mlp_body(x_ref, w1_ref, b1_ref, w2_ref, b2_ref, wo_ref, z_ref):
    h = jnp.maximum(_dot(x_ref[...], w1_ref[...]) + b1_ref[...], 0.0)
    h = jnp.maximum(_dot(h, w2_ref[...]) + b2_ref[...], 0.0)
    z_ref[...] = _dot(h, wo_ref[...])


def _mlp_project(x, W1, b1, W2, b2, Wo):
    grid = N_NODES // MLP_BLK
    return pl.pallas_call(
        _mlp_body,
        grid=(grid,),
        in_specs=[
            pl.BlockSpec((MLP_BLK, D_IN), lambda i: (i, 0)),
            pl.BlockSpec((D_IN, D_HID), lambda i: (0, 0)),
            pl.BlockSpec((1, D_HID), lambda i: (0, 0)),
            pl.BlockSpec((D_HID, D_HID), lambda i: (0, 0)),
            pl.BlockSpec((1, D_HID), lambda i: (0, 0)),
            pl.BlockSpec((D_HID, 1), lambda i: (0, 0)),
        ],
        out_specs=pl.BlockSpec((MLP_BLK, 1), lambda i: (i, 0)),
        out_shape=jax.ShapeDtypeStruct((N_NODES, 1), jnp.float32),
    )(x, W1, b1.reshape(1, D_HID), W2, b2.reshape(1, D_HID), Wo)


def _rsqrt16(d):
    bi = lax.bitcast_convert_type(d, jnp.int32)
    y = lax.bitcast_convert_type(jnp.int32(0x5F3759DF) - (bi >> 1),
                                 jnp.float32)
    for _ in range(3):
        y = y * (1.5 - 0.5 * d * y * y)
    return y


FK = 16


def _deg_body(dsts_hbm, dinv_hbm, agg_sh, dst_v, tmp_v, ones_v, ssem):
    tid = lax.axis_index("s")
    base = tid * NPT

    pltpu.sync_copy(dsts_hbm.at[pl.ds(tid * CPT, CPT)], dst_v)
    one = jnp.full((16,), 1.0, jnp.float32)
    for i in range(8):
        ones_v[pl.ds(i * 16, 16)] = one
    for i in range(N_VR):
        tmp_v[pl.ds(i * 16, 16)] = one
    pltpu.sync_copy(tmp_v, agg_sh.at[pl.ds(base, NPT)])
    plsc.subcore_barrier()

    @pl.loop(0, CPT, step=FK)
    def _deg(c0):
        for b in range(FK):
            pltpu.async_copy(ones_v, agg_sh.at[dst_v.at[c0 + b]], ssem,
                             add=True)

    pltpu.make_async_copy(dsts_hbm.at[pl.ds(tid * CPT, CPT)], dst_v,
                          ssem).wait()

    plsc.subcore_barrier()
    pltpu.sync_copy(agg_sh.at[pl.ds(base, NPT)], tmp_v)
    for i in range(N_VR):
        sl = pl.ds(i * 16, 16)
        tmp_v[sl] = _rsqrt16(tmp_v[sl])
    pltpu.sync_copy(tmp_v, dinv_hbm.at[pl.ds(base, NPT)])


def _prop_body(z0_hbm, srcs_hbm, dsts_hbm, dinv_hbm, bo_hbm, out_hbm,
               u_sh, agg_sh, src_v, dst_v, z0_v, z_v, dinv_v, tmp_v,
               vals_v, ones_v, bo_v, gsem, ssem):
    tid = lax.axis_index("s")
    base = tid * NPT

    pltpu.sync_copy(srcs_hbm.at[pl.ds(tid * CPT, CPT)], src_v)
    pltpu.sync_copy(dsts_hbm.at[pl.ds(tid * CPT, CPT)], dst_v)
    pltpu.sync_copy(z0_hbm.at[pl.ds(base, NPT)], z0_v)
    pltpu.sync_copy(dinv_hbm.at[pl.ds(base, NPT)], dinv_v)
    pltpu.sync_copy(bo_hbm, bo_v)

    for i in range(N_VR):
        sl = pl.ds(i * 16, 16)
        z_v[sl] = z0_v[sl]

    for _ in range(K_PROP):
        for i in range(N_VR):
            sl = pl.ds(i * 16, 16)
            tmp_v[sl] = dinv_v[sl] * z_v[sl]
        pltpu.sync_copy(tmp_v, u_sh.at[pl.ds(base, NPT)])
        pltpu.sync_copy(tmp_v, agg_sh.at[pl.ds(base, NPT)])
        plsc.subcore_barrier()

        @pl.loop(0, CPT, step=FK)
        def _edges(c0):
            gs = [pltpu.async_copy(u_sh.at[src_v.at[c0 + b]],
                                   vals_v.at[c0 + b], gsem)
                  for b in range(FK)]
            for d in gs:
                d.wait()
            for b in range(FK):
                pltpu.async_copy(vals_v.at[c0 + b],
                                 agg_sh.at[dst_v.at[c0 + b]], ssem,
                                 add=True)

        pltpu.make_async_copy(srcs_hbm.at[pl.ds(tid * CPT, CPT)], src_v,
                              ssem).wait()

        plsc.subcore_barrier()
        pltpu.sync_copy(agg_sh.at[pl.ds(base, NPT)], tmp_v)
        for i in range(N_VR):
            sl = pl.ds(i * 16, 16)
            z_v[sl] = ((1.0 - ALPHA) * dinv_v[sl] * tmp_v[sl]
                       + ALPHA * z0_v[sl])

    bo_vec = bo_v[...]
    for i in range(N_VR):
        sl = pl.ds(i * 16, 16)
        tmp_v[sl] = z_v[sl] + bo_vec
    pltpu.sync_copy(tmp_v, out_hbm.at[pl.ds(base, NPT)])


_deg = pl.kernel(
    _deg_body,
    out_type=jax.ShapeDtypeStruct((N_PAD,), jnp.float32),
    mesh=plsc.VectorSubcoreMesh(
        core_axis_name="c", subcore_axis_name="s", num_cores=1),
    scratch_types=[
        pltpu.VMEM_SHARED((N_PAD,), jnp.float32),
        pltpu.VMEM((CPT, CHUNK), jnp.int32),
        pltpu.VMEM((NPT,), jnp.float32),
        pltpu.VMEM((128,), jnp.float32),
        pltpu.SemaphoreType.DMA,
    ],
)

_prop = pl.kernel(
    _prop_body,
    out_type=jax.ShapeDtypeStruct((N_PAD,), jnp.float32),
    mesh=plsc.VectorSubcoreMesh(
        core_axis_name="c", subcore_axis_name="s", num_cores=1),
    scratch_types=[
        pltpu.VMEM_SHARED((N_PAD,), jnp.float32),
        pltpu.VMEM_SHARED((N_PAD,), jnp.float32),
        pltpu.VMEM((CPT, CHUNK), jnp.int32),
        pltpu.VMEM((CPT, CHUNK), jnp.int32),
        pltpu.VMEM((NPT,), jnp.float32),
        pltpu.VMEM((NPT,), jnp.float32),
        pltpu.VMEM((NPT,), jnp.float32),
        pltpu.VMEM((NPT,), jnp.float32),
        pltpu.VMEM((CPT, CHUNK), jnp.float32),
        pltpu.VMEM((128,), jnp.float32),
        pltpu.VMEM((16,), jnp.float32),
        pltpu.SemaphoreType.DMA,
        pltpu.SemaphoreType.DMA,
    ],
)


@jax.jit
def kernel(x, edge_index, W1, b1, W2, b2, Wo, bo):
    z0 = _mlp_project(x, W1, b1, W2, b2, Wo)[:, 0]
    z0p = jnp.pad(z0, (0, N_PAD - N_NODES))

    ei = edge_index.astype(jnp.int32)
    pad_idx = N_NODES + (jnp.arange(E_PAD - N_EDGES, dtype=jnp.int32)
                         % (N_PAD - N_NODES))
    srcs = jnp.concatenate([ei[0], pad_idx]).reshape(NT * CPT, CHUNK)
    dsts = jnp.concatenate([ei[1], pad_idx]).reshape(NT * CPT, CHUNK)
    bo16 = jnp.broadcast_to(bo.astype(jnp.float32), (16,))

    dinv = _deg(dsts)
    out = _prop(z0p, srcs, dsts, dinv, bo16)
    return out[:N_NODES]
